# baseline (speedup 1.0000x reference)
import functools

import jax
import jax.numpy as jnp
from jax import lax
from jax.experimental import pallas as pl
from jax.experimental.pallas import tpu as pltpu
from jax.experimental.pallas import tpu_sc as plsc

_D = 128
_BLK = 1024


def _rows(n):
    return (n + _BLK - 1) // _BLK


def _row_spec(width=_D):
    if width == 0:
        return pl.BlockSpec((_BLK,), lambda i: (i,))
    return pl.BlockSpec((_BLK, width), lambda i: (i, 0))


def _full_spec(shape):
    return pl.BlockSpec(shape, lambda i: tuple(0 for _ in shape))


def _emb2_body(t_ref, w1_ref, b1_ref, w2_ref, b2_ref, o_ref):
    t = t_ref[...]
    h = jnp.maximum(t[:, None] * w1_ref[...] + b1_ref[...], 0.0)
    o_ref[...] = jnp.dot(h, w2_ref[...], preferred_element_type=jnp.float32) + b2_ref[...]


def _bond_init_body(oh_ref, t_ref, w1_ref, b1_ref, w2_ref, b2_ref, w3_ref, b3_ref,
                    dw1_ref, db1_ref, dw2_ref, db2_ref, o_ref):
    h = jnp.maximum(jnp.dot(oh_ref[...], w1_ref[...], preferred_element_type=jnp.float32) + b1_ref[...], 0.0)
    h = jnp.maximum(jnp.dot(h, w2_ref[...], preferred_element_type=jnp.float32) + b2_ref[...], 0.0)
    y = jnp.dot(h, w3_ref[...], preferred_element_type=jnp.float32) + b3_ref[...]
    t = t_ref[...]
    g = jnp.maximum(t[:, None] * dw1_ref[...] + db1_ref[...], 0.0)
    o_ref[...] = y + jnp.dot(g, dw2_ref[...], preferred_element_type=jnp.float32) + db2_ref[...]


def _layer_body(a_ref, g_ref, w1a_ref, w1b_ref, b1_ref, w2_ref, b2_ref, w3_ref, b3_ref, o_ref):
    a = a_ref[...]
    h = (jnp.dot(a, w1a_ref[...], preferred_element_type=jnp.float32)
         + jnp.dot(g_ref[...], w1b_ref[...], preferred_element_type=jnp.float32)
         + b1_ref[...])
    h = jnp.maximum(h, 0.0)
    h = jnp.maximum(jnp.dot(h, w2_ref[...], preferred_element_type=jnp.float32) + b2_ref[...], 0.0)
    o_ref[...] = a + jnp.dot(h, w3_ref[...], preferred_element_type=jnp.float32) + b3_ref[...]


def _head_body(n, x_ref, t_ref, w1_ref, b1_ref, w2_ref, b2_ref, w3_ref, b3_ref, o_ref):
    h = jnp.maximum(jnp.dot(x_ref[...], w1_ref[...], preferred_element_type=jnp.float32) + b1_ref[...], 0.0)
    h = jnp.maximum(jnp.dot(h, w2_ref[...], preferred_element_type=jnp.float32) + b2_ref[...], 0.0)
    pred = jnp.dot(h, w3_ref[...], preferred_element_type=jnp.float32) + b3_ref[...]
    d = pred - t_ref[...][:, None]
    a = jnp.abs(d)
    hub = jnp.where(a < 1.0, 0.5 * d * d, a - 0.5)
    col = jax.lax.broadcasted_iota(jnp.int32, hub.shape, 1)
    row = (jax.lax.broadcasted_iota(jnp.int32, hub.shape, 0)
           + pl.program_id(0) * _BLK)
    hub = jnp.where((col == 0) & (row < n), hub, 0.0)
    o_ref[...] = jnp.sum(hub, axis=0, keepdims=True)[None]


def _emb2(t, p):
    n = t.shape[0]
    w1 = p[0]['W'][0]
    w2 = p[1]['W']
    return pl.pallas_call(
        _emb2_body,
        grid=(_rows(n),),
        in_specs=[_row_spec(0), _full_spec((_D,)), _full_spec((_D,)),
                  _full_spec((_D, _D)), _full_spec((_D,))],
        out_specs=_row_spec(),
        out_shape=jax.ShapeDtypeStruct((n, _D), jnp.float32),
    )(t, w1, p[0]['b'], w2, p[1]['b'])


def _bond_init(oh, t, p3, p2):
    n = oh.shape[0]
    w1 = jnp.zeros((_D, _D), jnp.float32).at[: p3[0]['W'].shape[0]].set(p3[0]['W'])
    return pl.pallas_call(
        _bond_init_body,
        grid=(_rows(n),),
        in_specs=[_row_spec(), _row_spec(0),
                  _full_spec((_D, _D)), _full_spec((_D,)),
                  _full_spec((_D, _D)), _full_spec((_D,)),
                  _full_spec((_D, _D)), _full_spec((_D,)),
                  _full_spec((_D,)), _full_spec((_D,)),
                  _full_spec((_D, _D)), _full_spec((_D,))],
        out_specs=_row_spec(),
        out_shape=jax.ShapeDtypeStruct((n, _D), jnp.float32),
    )(oh, t, w1, p3[0]['b'], p3[1]['W'], p3[1]['b'], p3[2]['W'], p3[2]['b'],
      p2[0]['W'][0], p2[0]['b'], p2[1]['W'], p2[1]['b'])


def _layer_mlp(a, g, p):
    n = a.shape[0]
    w1a = p[0]['W'][:_D]
    w1b = p[0]['W'][_D:]
    return pl.pallas_call(
        _layer_body,
        grid=(_rows(n),),
        in_specs=[_row_spec(), _row_spec(),
                  _full_spec((_D, _D)), _full_spec((_D, _D)), _full_spec((_D,)),
                  _full_spec((_D, _D)), _full_spec((_D,)),
                  _full_spec((_D, _D)), _full_spec((_D,))],
        out_specs=_row_spec(),
        out_shape=jax.ShapeDtypeStruct((n, _D), jnp.float32),
    )(a, g, w1a, w1b, p[0]['b'], p[1]['W'], p[1]['b'], p[2]['W'], p[2]['b'])


def _head_loss(x, t, p):
    n = x.shape[0]
    w3 = jnp.zeros((_D, _D), jnp.float32).at[:, :1].set(p[2]['W'])
    b3 = jnp.zeros((_D,), jnp.float32).at[0].set(p[2]['b'][0])
    partials = pl.pallas_call(
        functools.partial(_head_body, n),
        grid=(_rows(n),),
        in_specs=[_row_spec(), _row_spec(0),
                  _full_spec((_D, _D)), _full_spec((_D,)),
                  _full_spec((_D, _D)), _full_spec((_D,)),
                  _full_spec((_D, _D)), _full_spec((_D,))],
        out_specs=pl.BlockSpec((1, 1, _D), lambda i: (i, 0, 0)),
        out_shape=jax.ShapeDtypeStruct((_rows(n), 1, _D), jnp.float32),
    )(x, t, p[0]['W'], p[0]['b'], p[1]['W'], p[1]['b'], w3, b3)
    return jnp.sum(partials) / n


_NC, _NS = 2, 16
_L = 16
_CHUNK = 4096
_SHIFT, _MASK = 12, _CHUNK - 1
_SCAN = 2048
_B = 128


def _bin_body(e_total, nchunk, cap, src_h, dst_h, bd_h, bs_h, be_h, cnt_h,
              scan_dst, scan_src, stg_d, stg_s, stg_e, tmpv, cnt_sm):
    cid = lax.axis_index("c")
    sid = lax.axis_index("s")
    wid = sid * _NC + cid
    nw = _NC * _NS
    stripe = -(-(-(-e_total // nw)) // 8) * 8
    n_win = (stripe + _SCAN - 1) // _SCAN
    lane = jnp.arange(_L, dtype=jnp.int32)

    def _zc(i, _):
        cnt_sm[i] = jnp.int32(0)
        return 0
    lax.fori_loop(0, nchunk, _zc, 0)

    s_lo = wid * stripe
    s_hi = jnp.minimum(s_lo + stripe, e_total)

    def _win(w, _):
        nominal = s_lo + w * _SCAN
        start = pl.multiple_of(jnp.minimum(nominal, e_total - _SCAN), 8)
        pltpu.sync_copy(dst_h.at[pl.ds(start, _SCAN)], scan_dst)
        pltpu.sync_copy(src_h.at[pl.ds(start, _SCAN)], scan_src)
        lo = nominal - start
        hi = jnp.minimum(s_hi - start, _SCAN)

        def _vec(v, _):
            dstv = scan_dst[pl.ds(v * _L, _L)]
            srcv = scan_src[pl.ds(v * _L, _L)]
            for j in range(_L):
                idx = v * _L + j

                @pl.when((idx >= lo) & (idx < hi))
                def _():
                    d = dstv[j]
                    s = srcv[j]
                    c = lax.shift_right_logical(d, _SHIFT)
                    off = lax.bitwise_and(d, _MASK)
                    cnt = cnt_sm[c]
                    pos = lax.bitwise_and(cnt, 15)
                    vd = stg_d[pl.ds(c * _L, _L)]
                    stg_d[pl.ds(c * _L, _L)] = jnp.where(lane == pos, off, vd)
                    vs = stg_s[pl.ds(c * _L, _L)]
                    stg_s[pl.ds(c * _L, _L)] = jnp.where(lane == pos, s, vs)
                    ve = stg_e[pl.ds(c * _L, _L)]
                    stg_e[pl.ds(c * _L, _L)] = jnp.where(
                        lane == pos, start + idx, ve)

                    @pl.when(pos == 15)
                    def _():
                        bb = pl.multiple_of(
                            (wid * nchunk + c) * cap + cnt - 15, 16)
                        pltpu.sync_copy(stg_d.at[pl.ds(c * _L, _L)],
                                        bd_h.at[pl.ds(bb, _L)])
                        pltpu.sync_copy(stg_s.at[pl.ds(c * _L, _L)],
                                        bs_h.at[pl.ds(bb, _L)])
                        pltpu.sync_copy(stg_e.at[pl.ds(c * _L, _L)],
                                        be_h.at[pl.ds(bb, _L)])

                    cnt_sm[c] = cnt + 1
            return 0

        lax.fori_loop(0, _SCAN // _L, _vec, 0)
        return 0

    lax.fori_loop(0, n_win, _win, 0)

    def _fin(g, _):
        cnt = cnt_sm[g]

        @pl.when(lax.bitwise_and(cnt, 15) > 0)
        def _():
            bb = pl.multiple_of(
                (wid * nchunk + g) * cap + lax.bitwise_and(cnt, ~15), 16)
            pltpu.sync_copy(stg_d.at[pl.ds(g * _L, _L)],
                            bd_h.at[pl.ds(bb, _L)])
            pltpu.sync_copy(stg_s.at[pl.ds(g * _L, _L)],
                            bs_h.at[pl.ds(bb, _L)])
            pltpu.sync_copy(stg_e.at[pl.ds(g * _L, _L)],
                            be_h.at[pl.ds(bb, _L)])

        tmpv[pl.ds(0, _L)] = jnp.zeros((_L,), jnp.int32) + cnt
        pltpu.sync_copy(
            tmpv.at[pl.ds(0, _L)],
            cnt_h.at[pl.ds(pl.multiple_of((wid * nchunk + g) * _L, 16), _L)])
        return 0

    lax.fori_loop(0, nchunk, _fin, 0)


def _acc_body(nchunk, cap, table, feat, bd_h, bs_h, be_h, cnt_h, out_h,
              idx_d, idx_s, idx_e, cvbuf, rows_t, rows_f, zbuf,
              spmem, sem1, sem2):
    cid = lax.axis_index("c")
    sid = lax.axis_index("s")
    lane = jnp.arange(_L, dtype=jnp.int32)
    share = _CHUNK // _NS

    def _zrow(r, _):
        for j in range(_D // _L):
            zbuf[r, pl.ds(j * _L, _L)] = jnp.zeros((_L,), jnp.float32)
        return 0
    lax.fori_loop(0, _B, _zrow, 0)

    for b in range(share // _B):
        pltpu.sync_copy(zbuf, spmem.at[pl.ds(sid * share + b * _B, _B)])

    @pl.when(sid == 0)
    def _():
        pltpu.sync_copy(zbuf, spmem.at[pl.ds(_CHUNK, _B)])

    def _chunk(kk, _):
        g = cid + _NC * kk

        @pl.when(g < nchunk)
        def _():
            plsc.subcore_barrier()

            for half in range(2):
                b = sid * 2 + half
                pltpu.sync_copy(
                    cnt_h.at[pl.ds(
                        pl.multiple_of((b * nchunk + g) * _L, 16), _L)],
                    cvbuf)
                cnt = cvbuf[pl.ds(0, _L)][0]
                nb = lax.shift_right_logical(cnt + (_B - 1), 7)

                def _batch(bb, _):
                    boff = pl.multiple_of(
                        (b * nchunk + g) * cap + bb * _B, _B)
                    pltpu.sync_copy(bd_h.at[pl.ds(boff, _B)], idx_d)
                    pltpu.sync_copy(bs_h.at[pl.ds(boff, _B)], idx_s)
                    pltpu.sync_copy(be_h.at[pl.ds(boff, _B)], idx_e)
                    rem = cnt - bb * _B
                    for v in range(_B // _L):
                        m = (v * _L + lane) < rem
                        dv = idx_d[pl.ds(v * _L, _L)]
                        idx_d[pl.ds(v * _L, _L)] = jnp.where(
                            m, dv, _CHUNK + lane)
                        sv = idx_s[pl.ds(v * _L, _L)]
                        idx_s[pl.ds(v * _L, _L)] = jnp.where(m, sv, 0)
                        ev = idx_e[pl.ds(v * _L, _L)]
                        idx_e[pl.ds(v * _L, _L)] = jnp.where(m, ev, 0)
                    cp1 = pltpu.async_copy(table.at[idx_s], rows_t, sem1)
                    cp2 = pltpu.async_copy(feat.at[idx_e], rows_f, sem2)
                    cp1.wait()
                    cp2.wait()
                    pltpu.sync_copy(rows_t, spmem.at[idx_d], add=True)
                    pltpu.sync_copy(rows_f, spmem.at[idx_d], add=True)
                    return 0

                lax.fori_loop(0, nb, _batch, 0)

            plsc.subcore_barrier()

            for b in range(share // _B):
                row0 = sid * share + b * _B
                pltpu.sync_copy(
                    spmem.at[pl.ds(row0, _B)],
                    out_h.at[pl.ds(g * _CHUNK + row0, _B)])
                pltpu.sync_copy(zbuf, spmem.at[pl.ds(row0, _B)])

        return 0

    lax.fori_loop(0, (nchunk + _NC - 1) // _NC, _chunk, 0)


def _sc_segsum(table, feat, edges, n_out):
    e_total = feat.shape[0]
    nchunk = (n_out + _CHUNK - 1) // _CHUNK
    npad = nchunk * _CHUNK
    nw = _NC * _NS
    stripe = -(-(-(-e_total // nw)) // 8) * 8
    cap = ((stripe + _B - 1) // _B) * _B
    mesh = plsc.VectorSubcoreMesh(core_axis_name="c", subcore_axis_name="s")

    bin_k = pl.kernel(
        functools.partial(_bin_body, e_total, nchunk, cap),
        out_type=(
            jax.ShapeDtypeStruct((nw * nchunk * cap,), jnp.int32),
            jax.ShapeDtypeStruct((nw * nchunk * cap,), jnp.int32),
            jax.ShapeDtypeStruct((nw * nchunk * cap,), jnp.int32),
            jax.ShapeDtypeStruct((nw * nchunk * _L,), jnp.int32),
        ),
        mesh=mesh,
        scratch_types=[
            pltpu.VMEM((_SCAN,), jnp.int32),
            pltpu.VMEM((_SCAN,), jnp.int32),
            pltpu.VMEM((nchunk * _L,), jnp.int32),
            pltpu.VMEM((nchunk * _L,), jnp.int32),
            pltpu.VMEM((nchunk * _L,), jnp.int32),
            pltpu.VMEM((_L,), jnp.int32),
            pltpu.SMEM((64,), jnp.int32),
        ],
    )
    src = edges[:, 0] + jnp.zeros((), jnp.int32)
    dst = edges[:, 1] + jnp.zeros((), jnp.int32)
    bd, bs, be, cnts = bin_k(src, dst)

    acc_k = pl.kernel(
        functools.partial(_acc_body, nchunk, cap),
        out_type=jax.ShapeDtypeStruct((npad, _D), jnp.float32),
        mesh=mesh,
        scratch_types=[
            pltpu.VMEM((_B,), jnp.int32),
            pltpu.VMEM((_B,), jnp.int32),
            pltpu.VMEM((_B,), jnp.int32),
            pltpu.VMEM((_L,), jnp.int32),
            pltpu.VMEM((_B, _D), jnp.float32),
            pltpu.VMEM((_B, _D), jnp.float32),
            pltpu.VMEM((_B, _D), jnp.float32),
            pltpu.VMEM_SHARED((_CHUNK + _B, _D), jnp.float32),
            pltpu.SemaphoreType.DMA,
            pltpu.SemaphoreType.DMA,
        ],
    )
    return acc_k(table, feat, bd, bs, be, cnts)


def kernel(AtomBondGraph_edges, BondAngleGraph_edges, AngleDihedralGraph_edges,
           pos, x, bond_attr, bond_lengths, bond_angles, dihedral_angles,
           num_atoms, num_bonds, num_angles, num_graphs, atom_batch, params):
    del AtomBondGraph_edges, pos, x, num_atoms, num_bonds, num_angles, num_graphs, atom_batch
    BA = BondAngleGraph_edges
    AD = AngleDihedralGraph_edges
    n_bonds = bond_lengths.shape[0]
    n_angles = bond_angles.shape[0]

    vocab = [7, 5, 2]
    offs = [0, 7, 12]
    oh = jnp.zeros((n_bonds, _D), jnp.float32)
    for i, (v, o) in enumerate(zip(vocab, offs)):
        oh = oh + (jax.lax.broadcasted_iota(jnp.int32, (n_bonds, _D), 1)
                   == (bond_attr[:, i] + o)[:, None]).astype(jnp.float32)

    bond_h = _bond_init(oh, bond_lengths, params['bond_init'], params['dis_emb'])
    angle_h = _emb2(bond_angles, params['angle_emb'])
    dih_h = _emb2(dihedral_angles, params['dihedral_emb'])

    for lp in params['layers']:
        agg_a = _sc_segsum(angle_h, dih_h, AD, n_angles)
        angle_h = _layer_mlp(angle_h, agg_a, lp['angle_mlp'])
        agg_b = _sc_segsum(bond_h, angle_h, BA, n_bonds)
        bond_h = _layer_mlp(bond_h, agg_b, lp['bond_mlp'])

    loss = _head_loss(bond_h, bond_lengths, params['Blr_mlp'])
    loss = loss + _head_loss(angle_h, bond_angles, params['Bar_mlp'])
    loss = loss + _head_loss(dih_h, dihedral_angles, params['Dar_mlp'])
    return loss
